# baseline (speedup 1.0000x reference)
import functools

import jax
import jax.numpy as jnp
from jax import lax
from jax.experimental import pallas as pl
from jax.experimental.pallas import tpu as pltpu
from jax.experimental.pallas import tpu_sc as plsc

N_NODES = 10000
N_EDGES = 320000
NC = 2
NS = 16
NW = NC * NS
E_PER_W = N_EDGES // NW
CHUNK = 64
N_CHUNKS = 160
PASS_CHUNKS = 32
E_PAD = N_CHUNKS * CHUNK - E_PER_W
N_ACC = 10048
ROW_OFF = 624
ROW_SPAN = 640


def _make_spmm(F):
  mesh = plsc.VectorSubcoreMesh(core_axis_name="c", subcore_axis_name="s")

  @functools.partial(
      pl.kernel,
      out_type=jax.ShapeDtypeStruct((NC, N_NODES, F), jnp.float32),
      mesh=mesh,
      scratch_types=[
          pltpu.VMEM((PASS_CHUNKS, CHUNK), jnp.int32),
          pltpu.VMEM((PASS_CHUNKS, CHUNK), jnp.int32),
          pltpu.VMEM((CHUNK, F), jnp.float32),
          pltpu.VMEM((CHUNK, F), jnp.float32),
          pltpu.VMEM((CHUNK, F), jnp.float32),
          pltpu.VMEM((CHUNK, F), jnp.float32),
          pltpu.VMEM_SHARED((N_ACC, F), jnp.float32),
          [pltpu.SemaphoreType.DMA] * 4,
          [pltpu.SemaphoreType.DMA] * 4,
      ],
  )
  def spmm(table, src, dst, zeros, out, src_v, dst_v, rows0, rows1, rows2,
           rows3, acc, gsem, tsem):
    c = lax.axis_index("c")
    s = lax.axis_index("s")
    wid = c * NS + s
    rows = (rows0, rows1, rows2, rows3)
    row0 = pl.multiple_of(s * ROW_OFF, 8)
    pltpu.sync_copy(zeros, acc.at[pl.ds(row0, ROW_SPAN)])
    plsc.subcore_barrier()

    def gather(j, b):
      pltpu.async_copy(table.at[src_v.at[j]], rows[b], gsem[b])

    def wait_gather(j, b):
      pltpu.make_async_copy(table.at[src_v.at[j]], rows[b], gsem[b]).wait()

    def scatter(j, b):
      pltpu.async_copy(rows[b], acc.at[dst_v.at[j]], tsem[b], add=True)

    def wait_scatter(b):
      pltpu.make_async_copy(rows[b], acc.at[dst_v.at[0]], tsem[b]).wait()

    for p in range(N_CHUNKS // PASS_CHUNKS):
      pltpu.sync_copy(src.at[wid, pl.ds(p * PASS_CHUNKS, PASS_CHUNKS)], src_v)
      pltpu.sync_copy(dst.at[wid, pl.ds(p * PASS_CHUNKS, PASS_CHUNKS)], dst_v)
      for b in range(4):
        gather(b, b)
      for b in range(4):
        wait_gather(b, b)
        scatter(b, b)
        if b >= 2:
          wait_scatter(b - 2)
          gather(b + 2, b - 2)

      def body(k, carry):
        j = 4 * k
        for b in range(4):
          wait_gather(j + b, b)
          scatter(j + b, b)
          bb = (b + 2) % 4
          wait_scatter(bb)
          gather(j + b + 2, bb)
        return carry

      lax.fori_loop(1, PASS_CHUNKS // 4 - 1, body, 0)
      j = PASS_CHUNKS - 4
      for b in range(4):
        wait_gather(j + b, b)
        scatter(j + b, b)
        if b < 2:
          wait_scatter(b + 2)
          gather(j + b + 2, b + 2)
      for b in range(4):
        wait_scatter(b)
    plsc.subcore_barrier()
    pltpu.sync_copy(acc.at[pl.ds(row0, ROW_SPAN)],
                    out.at[c, pl.ds(row0, ROW_SPAN)])

  return spmm


_spmm_128 = _make_spmm(128)


def _tc1_body(x_ref, w_ref, out_ref):
  out_ref[...] = jnp.dot(x_ref[...], w_ref[...],
                         preferred_element_type=jnp.float32)


def _tc2_body(p_ref, b_ref, out_ref):
  out_ref[...] = jnp.maximum(p_ref[0] + p_ref[1] + b_ref[...], 0.0)


def _tc3_body(p_ref, w_ref, b_ref, out_ref):
  o = jnp.dot(p_ref[0] + p_ref[1], w_ref[...],
              preferred_element_type=jnp.float32) + b_ref[...]
  m = jnp.max(o, axis=1, keepdims=True)
  x = o - m
  lse = jnp.log(jnp.sum(jnp.exp(x), axis=1, keepdims=True))
  out_ref[...] = x - lse


def kernel(X, edge_index, W1, B1, W2, B2):
  wids = jnp.arange(NW, dtype=jnp.int32)[:, None]
  pad_ar = jnp.arange(E_PAD, dtype=jnp.int32)[None, :]
  src = jnp.concatenate(
      [edge_index[0].astype(jnp.int32).reshape(NW, E_PER_W),
       (wids * 313 + pad_ar * 41) % N_NODES],
      axis=1).reshape(NW, N_CHUNKS, CHUNK)
  dst = jnp.concatenate(
      [edge_index[1].astype(jnp.int32).reshape(NW, E_PER_W),
       N_NODES + (wids % NS) * 3 + pad_ar % 3],
      axis=1).reshape(NW, N_CHUNKS, CHUNK)

  s1 = pl.pallas_call(
      _tc1_body,
      out_shape=jax.ShapeDtypeStruct((N_NODES, 128), jnp.float32),
  )(X, W1)

  zeros = jnp.zeros((ROW_SPAN, 128), jnp.float32)
  p1 = _spmm_128(s1, src, dst, zeros)

  h = pl.pallas_call(
      _tc2_body,
      out_shape=jax.ShapeDtypeStruct((N_NODES, 128), jnp.float32),
  )(p1, B1.reshape(1, 128))

  p2 = _spmm_128(h, src, dst, zeros)

  return pl.pallas_call(
      _tc3_body,
      out_shape=jax.ShapeDtypeStruct((N_NODES, 64), jnp.float32),
  )(p2, W2, B2.reshape(1, 64))


# speedup vs baseline: 1.1947x; 1.1947x over previous
import functools

import jax
import jax.numpy as jnp
from jax import lax
from jax.experimental import pallas as pl
from jax.experimental.pallas import tpu as pltpu
from jax.experimental.pallas import tpu_sc as plsc

N_NODES = 10000
N_EDGES = 320000
NC = 2
NS = 16
NW = NC * NS
E_PER_W = N_EDGES // NW
CHUNK = 80
N_CHUNKS = 128
PASS_CHUNKS = 32
E_PAD = N_CHUNKS * CHUNK - E_PER_W
N_ACC = 10048
ROW_OFF = 624
ROW_SPAN = 640


def _make_spmm(F):
  mesh = plsc.VectorSubcoreMesh(core_axis_name="c", subcore_axis_name="s")

  @functools.partial(
      pl.kernel,
      out_type=jax.ShapeDtypeStruct((NC, N_NODES, F), jnp.float32),
      mesh=mesh,
      scratch_types=[
          pltpu.VMEM((PASS_CHUNKS, CHUNK), jnp.int32),
          pltpu.VMEM((PASS_CHUNKS, CHUNK), jnp.int32),
          pltpu.VMEM((CHUNK, F), jnp.float32),
          pltpu.VMEM((CHUNK, F), jnp.float32),
          pltpu.VMEM((CHUNK, F), jnp.float32),
          pltpu.VMEM_SHARED((N_ACC, F), jnp.float32),
          pltpu.SemaphoreType.DMA,
          pltpu.SemaphoreType.DMA,
          pltpu.SemaphoreType.DMA,
      ],
  )
  def spmm(table, src, dst, zeros, out, src_v, dst_v, rows0, rows1, rows2,
           acc, sem0, sem1, sem2):
    c = lax.axis_index("c")
    s = lax.axis_index("s")
    wid = c * NS + s
    row0 = pl.multiple_of(s * ROW_OFF, 8)
    pltpu.sync_copy(zeros, acc.at[pl.ds(row0, ROW_SPAN)])
    plsc.subcore_barrier()

    for p in range(N_CHUNKS // PASS_CHUNKS):
      pltpu.sync_copy(src.at[wid, pl.ds(p * PASS_CHUNKS, PASS_CHUNKS)], src_v)
      pltpu.sync_copy(dst.at[wid, pl.ds(p * PASS_CHUNKS, PASS_CHUNKS)], dst_v)
      pltpu.async_copy(table.at[src_v.at[0]], rows0, sem0)
      pltpu.async_copy(table.at[src_v.at[1]], rows1, sem1)

      def body(k, carry):
        j = 3 * k
        pltpu.async_copy(table.at[src_v.at[j + 2]], rows2, sem2)
        pltpu.make_async_copy(table.at[src_v.at[j]], rows0, sem0).wait()
        pltpu.sync_copy(rows0, acc.at[dst_v.at[j]], add=True)
        pltpu.async_copy(table.at[src_v.at[j + 3]], rows0, sem0)
        pltpu.make_async_copy(table.at[src_v.at[j + 1]], rows1, sem1).wait()
        pltpu.sync_copy(rows1, acc.at[dst_v.at[j + 1]], add=True)
        pltpu.async_copy(table.at[src_v.at[j + 4]], rows1, sem1)
        pltpu.make_async_copy(table.at[src_v.at[j + 2]], rows2, sem2).wait()
        pltpu.sync_copy(rows2, acc.at[dst_v.at[j + 2]], add=True)
        return carry

      lax.fori_loop(0, (PASS_CHUNKS - 2) // 3, body, 0)
      pltpu.make_async_copy(table.at[src_v.at[PASS_CHUNKS - 2]],
                            rows0, sem0).wait()
      pltpu.sync_copy(rows0, acc.at[dst_v.at[PASS_CHUNKS - 2]], add=True)
      pltpu.make_async_copy(table.at[src_v.at[PASS_CHUNKS - 1]],
                            rows1, sem1).wait()
      pltpu.sync_copy(rows1, acc.at[dst_v.at[PASS_CHUNKS - 1]], add=True)
    plsc.subcore_barrier()
    pltpu.sync_copy(acc.at[pl.ds(row0, ROW_SPAN)],
                    out.at[c, pl.ds(row0, ROW_SPAN)])

  return spmm


_spmm_128 = _make_spmm(128)


def _tc1_body(x_ref, w_ref, out_ref):
  out_ref[...] = jnp.dot(x_ref[...], w_ref[...],
                         preferred_element_type=jnp.float32)


def _tc2_body(p_ref, b_ref, out_ref):
  out_ref[...] = jnp.maximum(p_ref[0] + p_ref[1] + b_ref[...], 0.0)


def _tc3_body(p_ref, w_ref, b_ref, out_ref):
  o = jnp.dot(p_ref[0] + p_ref[1], w_ref[...],
              preferred_element_type=jnp.float32) + b_ref[...]
  m = jnp.max(o, axis=1, keepdims=True)
  x = o - m
  lse = jnp.log(jnp.sum(jnp.exp(x), axis=1, keepdims=True))
  out_ref[...] = x - lse


def kernel(X, edge_index, W1, B1, W2, B2):
  wids = jnp.arange(NW, dtype=jnp.int32)[:, None]
  pad_ar = jnp.arange(E_PAD, dtype=jnp.int32)[None, :]
  src = jnp.concatenate(
      [edge_index[0].astype(jnp.int32).reshape(NW, E_PER_W),
       (wids * 313 + pad_ar * 41) % N_NODES],
      axis=1).reshape(NW, N_CHUNKS, CHUNK)
  dst = jnp.concatenate(
      [edge_index[1].astype(jnp.int32).reshape(NW, E_PER_W),
       N_NODES + (wids % NS) * 3 + pad_ar % 3],
      axis=1).reshape(NW, N_CHUNKS, CHUNK)

  s1 = pl.pallas_call(
      _tc1_body,
      out_shape=jax.ShapeDtypeStruct((N_NODES, 128), jnp.float32),
  )(X, W1)

  zeros = jnp.zeros((ROW_SPAN, 128), jnp.float32)
  p1 = _spmm_128(s1, src, dst, zeros)

  h = pl.pallas_call(
      _tc2_body,
      out_shape=jax.ShapeDtypeStruct((N_NODES, 128), jnp.float32),
  )(p1, B1.reshape(1, 128))

  p2 = _spmm_128(h, src, dst, zeros)

  return pl.pallas_call(
      _tc3_body,
      out_shape=jax.ShapeDtypeStruct((N_NODES, 64), jnp.float32),
  )(p2, W2, B2.reshape(1, 64))
